# baseline (speedup 1.0000x reference)
import functools

import jax
import jax.numpy as jnp
from jax import lax
from jax.experimental import pallas as pl
from jax.experimental.pallas import tpu as pltpu
from jax.experimental.pallas import tpu_sc as plsc

SRC = 2048
BATCH = 4
HIDDEN = 768
ROWS = SRC * BATCH
NC, NS = 2, 16
NW = NC * NS
R_PER_W = ROWS // NW
CHUNK = 64
NCHUNK = R_PER_W // CHUNK


def _gather_sc(table, ids_flat):
    mesh = plsc.VectorSubcoreMesh(core_axis_name="c", subcore_axis_name="s")

    @functools.partial(
        pl.kernel,
        mesh=mesh,
        out_type=jax.ShapeDtypeStruct((ROWS, HIDDEN), jnp.float32),
        scratch_types=[
            pltpu.VMEM((R_PER_W,), jnp.int32),
            pltpu.VMEM((CHUNK, HIDDEN), jnp.float32),
            pltpu.VMEM((CHUNK, HIDDEN), jnp.float32),
            pltpu.SemaphoreType.DMA,
            pltpu.SemaphoreType.DMA,
        ],
    )
    def gather_kernel(table_hbm, idx_hbm, out_hbm, idx_v, buf0, buf1,
                      sem0, sem1):
        wid = lax.axis_index("s") * NC + lax.axis_index("c")
        base = wid * R_PER_W
        bufs, sems = (buf0, buf1), (sem0, sem1)
        pltpu.sync_copy(idx_hbm.at[pl.ds(base, R_PER_W)], idx_v)
        cps = [None] * NCHUNK
        cps[0] = pltpu.async_copy(
            table_hbm.at[idx_v.at[pl.ds(0, CHUNK)]], buf0, sem0)
        for c in range(NCHUNK):
            cps[c].wait()
            if c + 1 < NCHUNK:
                cps[c + 1] = pltpu.async_copy(
                    table_hbm.at[idx_v.at[pl.ds((c + 1) * CHUNK, CHUNK)]],
                    bufs[(c + 1) % 2], sems[(c + 1) % 2])
            pltpu.sync_copy(bufs[c % 2],
                            out_hbm.at[pl.ds(base + c * CHUNK, CHUNK)])

    return gather_kernel(table, ids_flat)


BS_S = 128
BLK = BS_S * BATCH


def _tc_fused(tok_flat, pos3, w_t, b2, g2, bt2):
    def body(tok_ref, pos_ref, w_ref, b_ref, g_ref, bt_ref, out_ref):
        pos = pos_ref[...].reshape(BLK, HIDDEN)
        acc = jnp.dot(pos, w_ref[...], preferred_element_type=jnp.float32)
        p = 1.0 / (1.0 + jnp.exp(-(acc + b_ref[...])))
        e = tok_ref[...] + p
        mean = jnp.mean(e, axis=1, keepdims=True)
        cen = e - mean
        var = jnp.mean(cen * cen, axis=1, keepdims=True)
        res = cen * lax.rsqrt(var + 1e-5) * g_ref[...] + bt_ref[...]
        out_ref[...] = res.reshape(BS_S, BATCH, HIDDEN)

    return pl.pallas_call(
        body,
        grid=(SRC // BS_S,),
        in_specs=[
            pl.BlockSpec((BLK, HIDDEN), lambda i: (i, 0)),
            pl.BlockSpec((BS_S, BATCH, HIDDEN), lambda i: (i, 0, 0)),
            pl.BlockSpec((HIDDEN, HIDDEN), lambda i: (0, 0)),
            pl.BlockSpec((1, HIDDEN), lambda i: (0, 0)),
            pl.BlockSpec((1, HIDDEN), lambda i: (0, 0)),
            pl.BlockSpec((1, HIDDEN), lambda i: (0, 0)),
        ],
        out_specs=pl.BlockSpec((BS_S, BATCH, HIDDEN), lambda i: (i, 0, 0)),
        out_shape=jax.ShapeDtypeStruct((SRC, BATCH, HIDDEN), jnp.float32),
    )(tok_flat, pos3, w_t, b2, g2, bt2)


def kernel(input_ids, position_ids, W_tok, W_pd, b_pd, gamma, beta):
    ids_flat = input_ids.reshape(ROWS).astype(jnp.int32)
    tok_flat = _gather_sc(W_tok, ids_flat)
    return _tc_fused(
        tok_flat, position_ids, W_pd.T,
        b_pd.reshape(1, HIDDEN), gamma.reshape(1, HIDDEN),
        beta.reshape(1, HIDDEN),
    )
